# baseline (speedup 1.0000x reference)
import functools

import jax
import jax.numpy as jnp
from jax import lax
from jax.experimental import pallas as pl
from jax.experimental.pallas import tpu as pltpu
from jax.experimental.pallas import tpu_sc as plsc

N = 5000
M = 5120
B = 256
NB = M // B
CH = 512
NCH = M // CH
IOU_THR = 0.5
_F = (1.0 + IOU_THR) / IOU_THR

D = 128
V = 5008

_NC = 2
_NS = 16
_NW = _NC * _NS
_BPW = M // _NW
_IC = 2
_ICW = _BPW // _IC


def _sc_gather_body(tab_hbm, idx_hbm, out_hbm, idx_v, rows_v, sem):
    wid = lax.axis_index("s") * _NC + lax.axis_index("c")
    base = wid * _BPW
    pltpu.sync_copy(idx_hbm.at[pl.ds(wid * _IC, _IC)], idx_v)
    for j in range(_IC):
        pltpu.async_copy(tab_hbm.at[idx_v.at[j]],
                         rows_v.at[pl.ds(j * _ICW, _ICW)], sem).wait()
    pltpu.sync_copy(rows_v, out_hbm.at[pl.ds(base, _BPW)])


@functools.partial(
    pl.kernel,
    mesh=plsc.VectorSubcoreMesh(core_axis_name="c", subcore_axis_name="s"),
    out_type=jax.ShapeDtypeStruct((M, D), jnp.float32),
    scratch_types=[
        pltpu.VMEM((_IC, _ICW), jnp.int32),
        pltpu.VMEM((_BPW, D), jnp.float32),
        pltpu.SemaphoreType.DMA,
    ],
)
def _sc_gather(tab_hbm, idx_hbm, out_hbm, idx_v, rows_v, sem):
    _sc_gather_body(tab_hbm, idx_hbm, out_hbm, idx_v, rows_v, sem)


def _iou_mask(rx1, ry1, rx2, ry2, ra, cx1, cy1, cx2, cy2, ca):
    ltx = jnp.maximum(rx1, cx1)
    lty = jnp.maximum(ry1, cy1)
    rbx = jnp.minimum(rx2, cx2)
    rby = jnp.minimum(ry2, cy2)
    w = jnp.maximum(rbx - ltx, 0.0)
    h = rby - lty
    inter3 = (_F * w) * h
    sa = ra + ca
    return (inter3 > sa).astype(jnp.bfloat16)


def _nms_body(tab, keep_ref, cf):
    blk = pl.program_id(0)

    @pl.when(blk == 0)
    def _init():
        keep_ref[...] = jnp.ones((1, M), jnp.float32)
        t = jnp.transpose(tab[:, 0:8], (1, 0))
        cf[0:8, :] = t
        cf[5:6, :] = (t[2:3, :] - t[0:1, :]) * (t[3:4, :] - t[1:2, :])

    base = blk * B

    def rows(c):
        return tab[pl.ds(pl.multiple_of(base, 8), B), c:c + 1]

    rx1, ry1, rx2, ry2 = rows(0), rows(1), rows(2), rows(3)
    ra = (rx2 - rx1) * (ry2 - ry1)

    def cols(c, off, w):
        return cf[c:c + 1, pl.ds(pl.multiple_of(off, 128), w)]

    m = _iou_mask(rx1, ry1, rx2, ry2, ra,
                  cols(0, base, B), cols(1, base, B),
                  cols(2, base, B), cols(3, base, B), cols(5, base, B))
    rix = lax.broadcasted_iota(jnp.int32, (B, B), 0)
    cix = lax.broadcasted_iota(jnp.int32, (B, B), 1)
    m = jnp.where(rix < cix, m, jnp.bfloat16(0))

    init = keep_ref[:, pl.ds(pl.multiple_of(base, 128), B)]

    def cond(c):
        return jnp.logical_not(c[1])

    def body(c):
        k, _ = c
        sup = lax.dot_general(k.astype(jnp.bfloat16), m,
                              (((1,), (0,)), ((), ())),
                              preferred_element_type=jnp.float32)
        k2 = jnp.where(sup > 0.0, 0.0, init)
        return k2, jnp.all(k2 == k)

    k, _ = lax.while_loop(cond, body, (init, jnp.array(False)))
    keep_ref[:, pl.ds(pl.multiple_of(base, 128), B)] = k

    bnd = base + B
    kb = k.astype(jnp.bfloat16)

    def chunk(c, _):
        off = c * CH
        mt = _iou_mask(rx1, ry1, rx2, ry2, ra,
                       cols(0, off, CH), cols(1, off, CH),
                       cols(2, off, CH), cols(3, off, CH), cols(5, off, CH))
        sup = lax.dot_general(kb, mt, (((1,), (0,)), ((), ())),
                              preferred_element_type=jnp.float32)
        gcol = off + lax.broadcasted_iota(jnp.int32, (1, CH), 1)
        old = keep_ref[:, pl.ds(pl.multiple_of(off, 128), CH)]
        keep_ref[:, pl.ds(pl.multiple_of(off, 128), CH)] = jnp.where(
            (sup > 0.0) & (gcol >= bnd), 0.0, old)
        return 0

    lax.fori_loop((blk + 1) * B // CH, NCH, chunk, 0)


@jax.jit
def kernel(boxes, scores):
    order = jnp.argsort(-scores)
    order_p = jnp.concatenate(
        [order, jnp.full((M - N,), N, order.dtype)]).reshape(_NW * _IC, _ICW)
    tab0 = jnp.concatenate(
        [boxes, scores[:, None], jnp.zeros((N, D - 5), jnp.float32)], axis=1)
    tabv = jnp.concatenate(
        [tab0, jnp.zeros((V - N, D), jnp.float32)], axis=0)

    tabp = _sc_gather(tabv, order_p)

    keep = pl.pallas_call(
        _nms_body,
        grid=(NB,),
        in_specs=[pl.BlockSpec((M, D), lambda i: (0, 0))],
        out_specs=pl.BlockSpec((1, M), lambda i: (0, 0)),
        out_shape=jax.ShapeDtypeStruct((1, M), jnp.float32),
        scratch_shapes=[pltpu.VMEM((8, M), jnp.float32)],
    )(tabp)

    km = keep.reshape(M)[:N]
    return tabp[:N, :5] * km[:, None]


# speedup vs baseline: 1.0353x; 1.0353x over previous
import functools

import jax
import jax.numpy as jnp
from jax import lax
from jax.experimental import pallas as pl
from jax.experimental.pallas import tpu as pltpu
from jax.experimental.pallas import tpu_sc as plsc

N = 5000
M = 5120
B = 256
NB = M // B
CH = 1024
NCH = M // CH
IOU_THR = 0.5
_F = (1.0 + IOU_THR) / IOU_THR

D = 128
V = 5008

_NC = 2
_NS = 16
_NW = _NC * _NS
_BPW = M // _NW
_IC = 2
_ICW = _BPW // _IC


def _sc_gather_body(tab_hbm, idx_hbm, out_hbm, idx_v, rows_v, sem):
    wid = lax.axis_index("s") * _NC + lax.axis_index("c")
    base = wid * _BPW
    pltpu.sync_copy(idx_hbm.at[pl.ds(wid * _IC, _IC)], idx_v)
    for j in range(_IC):
        pltpu.async_copy(tab_hbm.at[idx_v.at[j]],
                         rows_v.at[pl.ds(j * _ICW, _ICW)], sem).wait()
    pltpu.sync_copy(rows_v, out_hbm.at[pl.ds(base, _BPW)])


@functools.partial(
    pl.kernel,
    mesh=plsc.VectorSubcoreMesh(core_axis_name="c", subcore_axis_name="s"),
    out_type=jax.ShapeDtypeStruct((M, D), jnp.float32),
    scratch_types=[
        pltpu.VMEM((_IC, _ICW), jnp.int32),
        pltpu.VMEM((_BPW, D), jnp.float32),
        pltpu.SemaphoreType.DMA,
    ],
)
def _sc_gather(tab_hbm, idx_hbm, out_hbm, idx_v, rows_v, sem):
    _sc_gather_body(tab_hbm, idx_hbm, out_hbm, idx_v, rows_v, sem)


def _iou_mask(rx1, ry1, rx2, ry2, ra, cx1, cy1, cx2, cy2, ca):
    ltx = jnp.maximum(rx1, cx1)
    lty = jnp.maximum(ry1, cy1)
    rbx = jnp.minimum(rx2, cx2)
    rby = jnp.minimum(ry2, cy2)
    w = jnp.maximum(rbx - ltx, 0.0)
    h = rby - lty
    inter3 = (_F * w) * h
    sa = ra + ca
    return (inter3 > sa).astype(jnp.bfloat16)


def _nms_body(tab, keep_ref, cf):
    blk = pl.program_id(0)

    @pl.when(blk == 0)
    def _init():
        keep_ref[...] = jnp.ones((1, M), jnp.float32)
        t = jnp.transpose(tab[:, 0:8], (1, 0))
        cf[0:8, :] = t
        cf[5:6, :] = (t[2:3, :] - t[0:1, :]) * (t[3:4, :] - t[1:2, :])

    base = blk * B

    def rows(c):
        return tab[pl.ds(pl.multiple_of(base, 8), B), c:c + 1]

    rx1, ry1, rx2, ry2 = rows(0), rows(1), rows(2), rows(3)
    ra = (rx2 - rx1) * (ry2 - ry1)

    def cols(c, off, w):
        return cf[c:c + 1, pl.ds(pl.multiple_of(off, 128), w)]

    m = _iou_mask(rx1, ry1, rx2, ry2, ra,
                  cols(0, base, B), cols(1, base, B),
                  cols(2, base, B), cols(3, base, B), cols(5, base, B))
    rix = lax.broadcasted_iota(jnp.int32, (B, B), 0)
    cix = lax.broadcasted_iota(jnp.int32, (B, B), 1)
    m = jnp.where(rix < cix, m, jnp.bfloat16(0))

    init = keep_ref[:, pl.ds(pl.multiple_of(base, 128), B)]

    def cond(c):
        return jnp.logical_not(c[1])

    def body(c):
        k, _ = c
        sup = lax.dot_general(k.astype(jnp.bfloat16), m,
                              (((1,), (0,)), ((), ())),
                              preferred_element_type=jnp.float32)
        k2 = jnp.where(sup > 0.0, 0.0, init)
        return k2, jnp.all(k2 == k)

    k, _ = lax.while_loop(cond, body, (init, jnp.array(False)))
    keep_ref[:, pl.ds(pl.multiple_of(base, 128), B)] = k

    bnd = base + B
    kb = k.astype(jnp.bfloat16)

    def chunk(c, _):
        off = c * CH
        mt = _iou_mask(rx1, ry1, rx2, ry2, ra,
                       cols(0, off, CH), cols(1, off, CH),
                       cols(2, off, CH), cols(3, off, CH), cols(5, off, CH))
        sup = lax.dot_general(kb, mt, (((1,), (0,)), ((), ())),
                              preferred_element_type=jnp.float32)
        gcol = off + lax.broadcasted_iota(jnp.int32, (1, CH), 1)
        old = keep_ref[:, pl.ds(pl.multiple_of(off, 128), CH)]
        keep_ref[:, pl.ds(pl.multiple_of(off, 128), CH)] = jnp.where(
            (sup > 0.0) & (gcol >= bnd), 0.0, old)
        return 0

    lax.fori_loop((blk + 1) * B // CH, NCH, chunk, 0)


@jax.jit
def kernel(boxes, scores):
    order = jnp.argsort(-scores)
    order_p = jnp.concatenate(
        [order, jnp.full((M - N,), N, order.dtype)]).reshape(_NW * _IC, _ICW)
    tab0 = jnp.concatenate(
        [boxes, scores[:, None], jnp.zeros((N, D - 5), jnp.float32)], axis=1)
    tabv = jnp.concatenate(
        [tab0, jnp.zeros((V - N, D), jnp.float32)], axis=0)

    tabp = _sc_gather(tabv, order_p)

    keep = pl.pallas_call(
        _nms_body,
        grid=(NB,),
        in_specs=[pl.BlockSpec((M, D), lambda i: (0, 0))],
        out_specs=pl.BlockSpec((1, M), lambda i: (0, 0)),
        out_shape=jax.ShapeDtypeStruct((1, M), jnp.float32),
        scratch_shapes=[pltpu.VMEM((8, M), jnp.float32)],
    )(tabp)

    km = keep.reshape(M)[:N]
    return tabp[:N, :5] * km[:, None]


# speedup vs baseline: 1.1301x; 1.0916x over previous
import functools

import jax
import jax.numpy as jnp
from jax import lax
from jax.experimental import pallas as pl
from jax.experimental.pallas import tpu as pltpu
from jax.experimental.pallas import tpu_sc as plsc

N = 5000
M = 5120
B = 512
NB = M // B
CH = 1024
NCH = M // CH
IOU_THR = 0.5
_F = (1.0 + IOU_THR) / IOU_THR

D = 128
V = 5008

_NC = 2
_NS = 16
_NW = _NC * _NS
_BPW = M // _NW
_IC = 2
_ICW = _BPW // _IC


def _sc_gather_body(tab_hbm, idx_hbm, out_hbm, idx_v, rows_v, sem):
    wid = lax.axis_index("s") * _NC + lax.axis_index("c")
    base = wid * _BPW
    pltpu.sync_copy(idx_hbm.at[pl.ds(wid * _IC, _IC)], idx_v)
    for j in range(_IC):
        pltpu.async_copy(tab_hbm.at[idx_v.at[j]],
                         rows_v.at[pl.ds(j * _ICW, _ICW)], sem).wait()
    pltpu.sync_copy(rows_v, out_hbm.at[pl.ds(base, _BPW)])


@functools.partial(
    pl.kernel,
    mesh=plsc.VectorSubcoreMesh(core_axis_name="c", subcore_axis_name="s"),
    out_type=jax.ShapeDtypeStruct((M, D), jnp.float32),
    scratch_types=[
        pltpu.VMEM((_IC, _ICW), jnp.int32),
        pltpu.VMEM((_BPW, D), jnp.float32),
        pltpu.SemaphoreType.DMA,
    ],
)
def _sc_gather(tab_hbm, idx_hbm, out_hbm, idx_v, rows_v, sem):
    _sc_gather_body(tab_hbm, idx_hbm, out_hbm, idx_v, rows_v, sem)


def _iou_mask(rx1, ry1, rx2, ry2, ra, cx1, cy1, cx2, cy2, ca):
    ltx = jnp.maximum(rx1, cx1)
    lty = jnp.maximum(ry1, cy1)
    rbx = jnp.minimum(rx2, cx2)
    rby = jnp.minimum(ry2, cy2)
    w = jnp.maximum(rbx - ltx, 0.0)
    h = rby - lty
    inter3 = (_F * w) * h
    sa = ra + ca
    return (inter3 > sa).astype(jnp.bfloat16)


def _nms_body(tab, keep_ref, cf):
    blk = pl.program_id(0)

    @pl.when(blk == 0)
    def _init():
        keep_ref[...] = jnp.ones((1, M), jnp.float32)
        t = jnp.transpose(tab[:, 0:8], (1, 0))
        cf[0:8, :] = t
        cf[5:6, :] = (t[2:3, :] - t[0:1, :]) * (t[3:4, :] - t[1:2, :])

    base = blk * B

    def rows(c):
        return tab[pl.ds(pl.multiple_of(base, 8), B), c:c + 1]

    rx1, ry1, rx2, ry2 = rows(0), rows(1), rows(2), rows(3)
    ra = (rx2 - rx1) * (ry2 - ry1)

    def cols(c, off, w):
        return cf[c:c + 1, pl.ds(pl.multiple_of(off, 128), w)]

    m = _iou_mask(rx1, ry1, rx2, ry2, ra,
                  cols(0, base, B), cols(1, base, B),
                  cols(2, base, B), cols(3, base, B), cols(5, base, B))
    rix = lax.broadcasted_iota(jnp.int32, (B, B), 0)
    cix = lax.broadcasted_iota(jnp.int32, (B, B), 1)
    m = jnp.where(rix < cix, m, jnp.bfloat16(0))

    init = keep_ref[:, pl.ds(pl.multiple_of(base, 128), B)]

    def cond(c):
        return jnp.logical_not(c[1])

    def body(c):
        k, _ = c
        sup = lax.dot_general(k.astype(jnp.bfloat16), m,
                              (((1,), (0,)), ((), ())),
                              preferred_element_type=jnp.float32)
        k2 = jnp.where(sup > 0.0, 0.0, init)
        return k2, jnp.all(k2 == k)

    k, _ = lax.while_loop(cond, body, (init, jnp.array(False)))
    keep_ref[:, pl.ds(pl.multiple_of(base, 128), B)] = k

    bnd = base + B
    kb = k.astype(jnp.bfloat16)

    def chunk(c, _):
        off = c * CH
        mt = _iou_mask(rx1, ry1, rx2, ry2, ra,
                       cols(0, off, CH), cols(1, off, CH),
                       cols(2, off, CH), cols(3, off, CH), cols(5, off, CH))
        sup = lax.dot_general(kb, mt, (((1,), (0,)), ((), ())),
                              preferred_element_type=jnp.float32)
        gcol = off + lax.broadcasted_iota(jnp.int32, (1, CH), 1)
        old = keep_ref[:, pl.ds(pl.multiple_of(off, 128), CH)]
        keep_ref[:, pl.ds(pl.multiple_of(off, 128), CH)] = jnp.where(
            (sup > 0.0) & (gcol >= bnd), 0.0, old)
        return 0

    lax.fori_loop((blk + 1) * B // CH, NCH, chunk, 0)


@jax.jit
def kernel(boxes, scores):
    order = jnp.argsort(-scores)
    order_p = jnp.concatenate(
        [order, jnp.full((M - N,), N, order.dtype)]).reshape(_NW * _IC, _ICW)
    tab0 = jnp.concatenate(
        [boxes, scores[:, None], jnp.zeros((N, D - 5), jnp.float32)], axis=1)
    tabv = jnp.concatenate(
        [tab0, jnp.zeros((V - N, D), jnp.float32)], axis=0)

    tabp = _sc_gather(tabv, order_p)

    keep = pl.pallas_call(
        _nms_body,
        grid=(NB,),
        in_specs=[pl.BlockSpec((M, D), lambda i: (0, 0))],
        out_specs=pl.BlockSpec((1, M), lambda i: (0, 0)),
        out_shape=jax.ShapeDtypeStruct((1, M), jnp.float32),
        scratch_shapes=[pltpu.VMEM((8, M), jnp.float32)],
    )(tabp)

    km = keep.reshape(M)[:N]
    return tabp[:N, :5] * km[:, None]


# speedup vs baseline: 1.1479x; 1.0157x over previous
import functools

import jax
import jax.numpy as jnp
from jax import lax
from jax.experimental import pallas as pl
from jax.experimental.pallas import tpu as pltpu
from jax.experimental.pallas import tpu_sc as plsc

N = 5000
M = 5120
B = 1024
NB = M // B
CH = 1024
NCH = M // CH
IOU_THR = 0.5
_F = (1.0 + IOU_THR) / IOU_THR

D = 128
V = 5008

_NC = 2
_NS = 16
_NW = _NC * _NS
_BPW = M // _NW
_IC = 2
_ICW = _BPW // _IC


def _sc_gather_body(tab_hbm, idx_hbm, out_hbm, idx_v, rows_v, sem):
    wid = lax.axis_index("s") * _NC + lax.axis_index("c")
    base = wid * _BPW
    pltpu.sync_copy(idx_hbm.at[pl.ds(wid * _IC, _IC)], idx_v)
    for j in range(_IC):
        pltpu.async_copy(tab_hbm.at[idx_v.at[j]],
                         rows_v.at[pl.ds(j * _ICW, _ICW)], sem).wait()
    pltpu.sync_copy(rows_v, out_hbm.at[pl.ds(base, _BPW)])


@functools.partial(
    pl.kernel,
    mesh=plsc.VectorSubcoreMesh(core_axis_name="c", subcore_axis_name="s"),
    out_type=jax.ShapeDtypeStruct((M, D), jnp.float32),
    scratch_types=[
        pltpu.VMEM((_IC, _ICW), jnp.int32),
        pltpu.VMEM((_BPW, D), jnp.float32),
        pltpu.SemaphoreType.DMA,
    ],
)
def _sc_gather(tab_hbm, idx_hbm, out_hbm, idx_v, rows_v, sem):
    _sc_gather_body(tab_hbm, idx_hbm, out_hbm, idx_v, rows_v, sem)


def _iou_mask(rx1, ry1, rx2, ry2, ra, cx1, cy1, cx2, cy2, ca):
    ltx = jnp.maximum(rx1, cx1)
    lty = jnp.maximum(ry1, cy1)
    rbx = jnp.minimum(rx2, cx2)
    rby = jnp.minimum(ry2, cy2)
    w = jnp.maximum(rbx - ltx, 0.0)
    h = rby - lty
    inter3 = (_F * w) * h
    sa = ra + ca
    return (inter3 > sa).astype(jnp.bfloat16)


def _nms_body(tab, keep_ref, cf):
    blk = pl.program_id(0)

    @pl.when(blk == 0)
    def _init():
        keep_ref[...] = jnp.ones((1, M), jnp.float32)
        t = jnp.transpose(tab[:, 0:8], (1, 0))
        cf[0:8, :] = t
        cf[5:6, :] = (t[2:3, :] - t[0:1, :]) * (t[3:4, :] - t[1:2, :])

    base = blk * B

    def rows(c):
        return tab[pl.ds(pl.multiple_of(base, 8), B), c:c + 1]

    rx1, ry1, rx2, ry2 = rows(0), rows(1), rows(2), rows(3)
    ra = (rx2 - rx1) * (ry2 - ry1)

    def cols(c, off, w):
        return cf[c:c + 1, pl.ds(pl.multiple_of(off, 128), w)]

    m = _iou_mask(rx1, ry1, rx2, ry2, ra,
                  cols(0, base, B), cols(1, base, B),
                  cols(2, base, B), cols(3, base, B), cols(5, base, B))
    rix = lax.broadcasted_iota(jnp.int32, (B, B), 0)
    cix = lax.broadcasted_iota(jnp.int32, (B, B), 1)
    m = jnp.where(rix < cix, m, jnp.bfloat16(0))

    init = keep_ref[:, pl.ds(pl.multiple_of(base, 128), B)]

    def cond(c):
        return jnp.logical_not(c[1])

    def body(c):
        k, _ = c
        sup = lax.dot_general(k.astype(jnp.bfloat16), m,
                              (((1,), (0,)), ((), ())),
                              preferred_element_type=jnp.float32)
        k2 = jnp.where(sup > 0.0, 0.0, init)
        return k2, jnp.all(k2 == k)

    k, _ = lax.while_loop(cond, body, (init, jnp.array(False)))
    keep_ref[:, pl.ds(pl.multiple_of(base, 128), B)] = k

    bnd = base + B
    kb = k.astype(jnp.bfloat16)

    def chunk(c, _):
        off = c * CH
        mt = _iou_mask(rx1, ry1, rx2, ry2, ra,
                       cols(0, off, CH), cols(1, off, CH),
                       cols(2, off, CH), cols(3, off, CH), cols(5, off, CH))
        sup = lax.dot_general(kb, mt, (((1,), (0,)), ((), ())),
                              preferred_element_type=jnp.float32)
        gcol = off + lax.broadcasted_iota(jnp.int32, (1, CH), 1)
        old = keep_ref[:, pl.ds(pl.multiple_of(off, 128), CH)]
        keep_ref[:, pl.ds(pl.multiple_of(off, 128), CH)] = jnp.where(
            (sup > 0.0) & (gcol >= bnd), 0.0, old)
        return 0

    lax.fori_loop((blk + 1) * B // CH, NCH, chunk, 0)


@jax.jit
def kernel(boxes, scores):
    order = jnp.argsort(-scores)
    order_p = jnp.concatenate(
        [order, jnp.full((M - N,), N, order.dtype)]).reshape(_NW * _IC, _ICW)
    tab0 = jnp.concatenate(
        [boxes, scores[:, None], jnp.zeros((N, D - 5), jnp.float32)], axis=1)
    tabv = jnp.concatenate(
        [tab0, jnp.zeros((V - N, D), jnp.float32)], axis=0)

    tabp = _sc_gather(tabv, order_p)

    keep = pl.pallas_call(
        _nms_body,
        grid=(NB,),
        in_specs=[pl.BlockSpec((M, D), lambda i: (0, 0))],
        out_specs=pl.BlockSpec((1, M), lambda i: (0, 0)),
        out_shape=jax.ShapeDtypeStruct((1, M), jnp.float32),
        scratch_shapes=[pltpu.VMEM((8, M), jnp.float32)],
    )(tabp)

    km = keep.reshape(M)[:N]
    return tabp[:N, :5] * km[:, None]
